# baseline (speedup 1.0000x reference)
import jax
import jax.numpy as jnp
from jax.experimental import pallas as pl
from jax.experimental.pallas import tpu as pltpu

_NDMA = 16


def _copy_body(w_ref, o_ref, sems):
    n_rows = w_ref.shape[0]
    slab = n_rows // _NDMA
    copies = [
        pltpu.make_async_copy(
            w_ref.at[pl.ds(i * slab, slab)],
            o_ref.at[pl.ds(i * slab, slab)],
            sems.at[i],
        )
        for i in range(_NDMA)
    ]
    for c in copies:
        c.start()
    for c in copies:
        c.wait()


def kernel(emb, W):
    n_ctx, n_embd = W.shape
    seq_len = emb.shape[1]
    out = pl.pallas_call(
        _copy_body,
        in_specs=[pl.BlockSpec(memory_space=pl.ANY)],
        out_specs=pl.BlockSpec(memory_space=pl.ANY),
        out_shape=jax.ShapeDtypeStruct((seq_len, n_embd), jnp.float32),
        scratch_shapes=[pltpu.SemaphoreType.DMA((_NDMA,))],
    )(W)
    return out[None, :, :]


# speedup vs baseline: 48.6410x; 48.6410x over previous
import jax
import jax.numpy as jnp
from jax.experimental import pallas as pl


def _copy_body(w_ref, o_ref):
    o_ref[...] = w_ref[...]


def kernel(emb, W):
    n_ctx, n_embd = W.shape
    seq_len = emb.shape[1]
    blk = 1024
    grid = seq_len // blk
    out = pl.pallas_call(
        _copy_body,
        grid=(grid,),
        in_specs=[pl.BlockSpec((blk, n_embd), lambda i: (i, 0))],
        out_specs=pl.BlockSpec((blk, n_embd), lambda i: (i, 0)),
        out_shape=jax.ShapeDtypeStruct((seq_len, n_embd), jnp.float32),
    )(W)
    return out[None, :, :]


# speedup vs baseline: 48.8209x; 1.0037x over previous
import jax
import jax.numpy as jnp
from jax.experimental import pallas as pl
from jax.experimental.pallas import tpu as pltpu

_CHUNK = 512
_NBUF = 8


def _copy_body(w_ref, o_ref, bufs, lsems, ssems):
    n = w_ref.shape[0] // _CHUNK

    def load(i):
        b = i % _NBUF
        return pltpu.make_async_copy(
            w_ref.at[pl.ds(i * _CHUNK, _CHUNK)], bufs.at[b], lsems.at[b])

    def store(i):
        b = i % _NBUF
        return pltpu.make_async_copy(
            bufs.at[b], o_ref.at[pl.ds(i * _CHUNK, _CHUNK)], ssems.at[b])

    waited = [False] * n
    for i in range(min(_NBUF - 1, n)):
        load(i).start()
    for i in range(n):
        load(i).wait()
        store(i).start()
        j = i + _NBUF - 1
        if j < n:
            if i >= 1 and not waited[i - 1]:
                store(i - 1).wait()
                waited[i - 1] = True
            load(j).start()
    for i in range(n):
        if not waited[i]:
            store(i).wait()


def kernel(emb, W):
    n_ctx, n_embd = W.shape
    seq_len = emb.shape[1]
    out = pl.pallas_call(
        _copy_body,
        in_specs=[pl.BlockSpec(memory_space=pl.ANY)],
        out_specs=pl.BlockSpec(memory_space=pl.ANY),
        out_shape=jax.ShapeDtypeStruct((seq_len, n_embd), jnp.float32),
        scratch_shapes=[
            pltpu.VMEM((_NBUF, _CHUNK, n_embd), jnp.float32),
            pltpu.SemaphoreType.DMA((_NBUF,)),
            pltpu.SemaphoreType.DMA((_NBUF,)),
        ],
    )(W)
    return out[None, :, :]
